# baseline (speedup 1.0000x reference)
import jax
import jax.numpy as jnp
from jax.experimental import pallas as pl
import jax.experimental.pallas.tpu as pltpu

N = 4096
NFEAT = 128
NHID = 32
NCLASS = 16
TILE = 512

_DEFAULT = jax.lax.Precision.DEFAULT

_W1_R0 = 0
_B1_R = NFEAT
_B2_R = NFEAT + 1
_W2_R0 = NFEAT + 8


def _dot(a, b):
    return jax.lax.dot_general(a, b, (((1,), (0,)), ((), ())),
                               precision=_DEFAULT,
                               preferred_element_type=jnp.float32)


def _gcn_kernel(x_ref, adj_ref, pk_ref, out_ref, sel_ref, h_ref):
    p = pl.program_id(0)
    t = pl.program_id(1)

    @pl.when((p == 0) & (t == 0))
    def _():
        sel_ref[...] = _dot(x_ref[...], pk_ref[_W1_R0:_W1_R0 + NFEAT, :])

    @pl.when((p == 1) & (t == 0))
    def _():
        sel_ref[...] = _dot(h_ref[...], pk_ref[_W2_R0:_W2_R0 + NHID, :])

    acc = _dot(adj_ref[0], sel_ref[...])

    @pl.when(p == 0)
    def _():
        h_ref[pl.ds(t * TILE, TILE), :] = jnp.maximum(
            acc + pk_ref[_B1_R:_B1_R + 1, :], 0.0)

    @pl.when(p == 1)
    def _():
        o = acc[:, :NCLASS] + pk_ref[_B2_R:_B2_R + 1, :NCLASS]
        out_ref[...] = jnp.minimum(o, 0.0) - jnp.log1p(jnp.exp(-jnp.abs(o)))


@jax.jit
def kernel(x, adj_list, W1, b1, W2, b2):
    pad = NHID - NCLASS
    packed = jnp.concatenate([
        W1,
        b1.reshape(1, NHID),
        jnp.pad(b2.reshape(1, NCLASS), ((0, 0), (0, pad))),
        jnp.zeros((6, NHID), jnp.float32),
        jnp.pad(W2, ((0, 0), (0, pad))),
    ], axis=0)

    grid = (2, N // TILE)
    return pl.pallas_call(
        _gcn_kernel,
        grid=grid,
        in_specs=[
            pl.BlockSpec((N, NFEAT), lambda p, t: (0, 0)),
            pl.BlockSpec((1, TILE, N), lambda p, t: (p, t, 0)),
            pl.BlockSpec(packed.shape, lambda p, t: (0, 0)),
        ],
        out_specs=pl.BlockSpec((TILE, NCLASS), lambda p, t: (p * t, 0)),
        out_shape=jax.ShapeDtypeStruct((N, NCLASS), jnp.float32),
        scratch_shapes=[
            pltpu.VMEM((N, NHID), jnp.float32),
            pltpu.VMEM((N, NHID), jnp.float32),
        ],
    )(x, adj_list, packed)
